# Initial kernel scaffold; baseline (speedup 1.0000x reference)
import functools

import jax
import jax.numpy as jnp
from jax import lax
from jax.experimental import pallas as pl
from jax.experimental.pallas import tpu as pltpu
from jax.experimental.pallas import tpu_sc as plsc

N_NODES = 10000
N_EDGES = 320000
DIM = 128

NC = 2
NS = 16
CHUNK = 128
CHUNKS_PER_TILE = 80
EDGES_PER_TILE = CHUNK * CHUNKS_PER_TILE
E_PAD = NC * NS * EDGES_PER_TILE
ACC_ROWS = N_NODES + NS
ROWS_PER_TILE_INIT = ACC_ROWS // NS
ROWS_PER_TILE_OUT = N_NODES // NS


def _sc_aggregate(x, src_t, dst_t, zeros_init):
  mesh = plsc.VectorSubcoreMesh(
      core_axis_name="c", subcore_axis_name="s", num_cores=NC,
      num_subcores=NS)

  @functools.partial(
      pl.kernel,
      out_type=jax.ShapeDtypeStruct((NC, N_NODES, DIM), jnp.float32),
      mesh=mesh,
      scratch_types=[
          pltpu.VMEM_SHARED((ACC_ROWS, DIM), jnp.float32),
          pltpu.VMEM((CHUNKS_PER_TILE, CHUNK), jnp.int32),
          pltpu.VMEM((CHUNKS_PER_TILE, CHUNK), jnp.int32),
          pltpu.VMEM((CHUNK, DIM), jnp.float32),
          pltpu.SemaphoreType.DMA,
      ],
  )
  def sc_kernel(x_hbm, src_hbm, dst_hbm, zer_hbm, out_hbm,
                acc, src_v, dst_v, rows, sem):
    c = lax.axis_index("c")
    s = lax.axis_index("s")
    pltpu.sync_copy(src_hbm.at[c, s], src_v)
    pltpu.sync_copy(dst_hbm.at[c, s], dst_v)
    pltpu.sync_copy(zer_hbm.at[pl.ds(s * ROWS_PER_TILE_INIT,
                                     ROWS_PER_TILE_INIT)],
                    acc.at[pl.ds(s * ROWS_PER_TILE_INIT,
                                 ROWS_PER_TILE_INIT)])
    plsc.subcore_barrier()

    @pl.loop(0, CHUNKS_PER_TILE)
    def _(i):
      pltpu.async_copy(x_hbm.at[src_v.at[i]], rows, sem).wait()
      pltpu.sync_copy(rows, acc.at[dst_v.at[i]], add=True)

    plsc.subcore_barrier()
    pltpu.sync_copy(
        acc.at[pl.ds(s * ROWS_PER_TILE_OUT, ROWS_PER_TILE_OUT)],
        out_hbm.at[c, pl.ds(s * ROWS_PER_TILE_OUT, ROWS_PER_TILE_OUT)])

  return sc_kernel(x, src_t, dst_t, zeros_init)


def _tc_body(a_ref, w1_ref, b1_ref, w2_ref, b2_ref, o_ref):
  a = a_ref[0] + a_ref[1]
  h = lax.dot_general(a, w1_ref[...], (((1,), (1,)), ((), ())),
                      preferred_element_type=jnp.float32)
  h = jnp.maximum(h + b1_ref[...], 0.0)
  o = lax.dot_general(w2_ref[...], h, (((1,), (1,)), ((), ())),
                      preferred_element_type=jnp.float32)
  o_ref[...] = o + b2_ref[...]


def _tc_linear(agg2, W1, b1, W2, b2):
  BLK = 1000
  grid = N_NODES // BLK
  return pl.pallas_call(
      _tc_body,
      out_shape=jax.ShapeDtypeStruct((DIM, N_NODES), jnp.float32),
      grid=(grid,),
      in_specs=[
          pl.BlockSpec((NC, BLK, DIM), lambda i: (0, i, 0)),
          pl.BlockSpec((DIM, DIM), lambda i: (0, 0)),
          pl.BlockSpec((1, DIM), lambda i: (0, 0)),
          pl.BlockSpec((DIM, DIM), lambda i: (0, 0)),
          pl.BlockSpec((DIM, 1), lambda i: (0, 0)),
      ],
      out_specs=pl.BlockSpec((DIM, BLK), lambda i: (0, i)),
  )(agg2, W1, b1.reshape(1, DIM), W2, b2.reshape(DIM, 1))


def kernel(x, edge_index, W1, b1, W2, b2):
  src = edge_index[0]
  dst = edge_index[1]
  pad = E_PAD - N_EDGES
  src_p = jnp.concatenate([src, jnp.zeros((pad,), jnp.int32)])
  dst_p = jnp.concatenate(
      [dst, jnp.full((pad,), N_NODES, jnp.int32)])
  src_t = src_p.reshape(NC, NS, CHUNKS_PER_TILE, CHUNK)
  dst_t = dst_p.reshape(NC, NS, CHUNKS_PER_TILE, CHUNK)
  zeros_init = jnp.zeros((ACC_ROWS, DIM), jnp.float32)
  agg2 = _sc_aggregate(x, src_t, dst_t, zeros_init)
  return _tc_linear(agg2, W1, b1, W2, b2)


# speedup vs baseline: 3.4178x; 3.4178x over previous
import functools

import jax
import jax.numpy as jnp
from jax import lax
from jax.experimental import pallas as pl
from jax.experimental.pallas import tpu as pltpu
from jax.experimental.pallas import tpu_sc as plsc

N_NODES = 10000
N_EDGES = 320000
DIM = 128

NC = 2
NS = 16
CHUNK = 128
CHUNKS_PER_TILE = 80
EDGES_PER_TILE = CHUNK * CHUNKS_PER_TILE
E_PAD = NC * NS * EDGES_PER_TILE
ACC_ROWS = 10112
ROWS_PER_TILE = ACC_ROWS // NS


def _sc_aggregate(x, src_t, dst_t, zeros_init):
  mesh = plsc.VectorSubcoreMesh(
      core_axis_name="c", subcore_axis_name="s", num_cores=NC,
      num_subcores=NS)

  @functools.partial(
      pl.kernel,
      out_type=jax.ShapeDtypeStruct((NC, ACC_ROWS, DIM), jnp.float32),
      mesh=mesh,
      scratch_types=[
          pltpu.VMEM_SHARED((ACC_ROWS, DIM), jnp.float32),
          pltpu.VMEM((CHUNKS_PER_TILE, CHUNK), jnp.int32),
          pltpu.VMEM((CHUNKS_PER_TILE, CHUNK), jnp.int32),
          pltpu.VMEM((CHUNK, DIM), jnp.float32),
          pltpu.SemaphoreType.DMA,
      ],
  )
  def sc_kernel(x_hbm, src_hbm, dst_hbm, zer_hbm, out_hbm,
                acc, src_v, dst_v, rows, sem):
    c = lax.axis_index("c")
    s = lax.axis_index("s")
    pltpu.sync_copy(src_hbm.at[c, s], src_v)
    pltpu.sync_copy(dst_hbm.at[c, s], dst_v)
    pltpu.sync_copy(zer_hbm.at[pl.ds(s * ROWS_PER_TILE, ROWS_PER_TILE)],
                    acc.at[pl.ds(s * ROWS_PER_TILE, ROWS_PER_TILE)])
    plsc.subcore_barrier()

    @pl.loop(0, CHUNKS_PER_TILE)
    def _(i):
      pltpu.async_copy(x_hbm.at[src_v.at[i]], rows, sem).wait()
      pltpu.sync_copy(rows, acc.at[dst_v.at[i]], add=True)

    plsc.subcore_barrier()
    pltpu.sync_copy(
        acc.at[pl.ds(s * ROWS_PER_TILE, ROWS_PER_TILE)],
        out_hbm.at[c, pl.ds(s * ROWS_PER_TILE, ROWS_PER_TILE)])

  return sc_kernel(x, src_t, dst_t, zeros_init)


def _tc_body(a_ref, w1_ref, b1_ref, w2_ref, b2_ref, o_ref):
  a = a_ref[0, :N_NODES] + a_ref[1, :N_NODES]
  h = lax.dot_general(a, w1_ref[...], (((1,), (1,)), ((), ())),
                      preferred_element_type=jnp.float32)
  h = jnp.maximum(h + b1_ref[...], 0.0)
  o = lax.dot_general(w2_ref[...], h, (((1,), (1,)), ((), ())),
                      preferred_element_type=jnp.float32)
  o_ref[...] = o + b2_ref[...]


def _tc_linear(agg2, W1, b1, W2, b2):
  return pl.pallas_call(
      _tc_body,
      out_shape=jax.ShapeDtypeStruct((DIM, N_NODES), jnp.float32),
  )(agg2, W1, b1.reshape(1, DIM), W2, b2.reshape(DIM, 1))


def kernel(x, edge_index, W1, b1, W2, b2):
  src = edge_index[0]
  dst = edge_index[1]
  pad = E_PAD - N_EDGES
  src_p = jnp.concatenate([src, jnp.zeros((pad,), jnp.int32)])
  dst_p = jnp.concatenate(
      [dst, jnp.full((pad,), N_NODES, jnp.int32)])
  src_t = src_p.reshape(NC, NS, CHUNKS_PER_TILE, CHUNK)
  dst_t = dst_p.reshape(NC, NS, CHUNKS_PER_TILE, CHUNK)
  zeros_init = jnp.zeros((ACC_ROWS, DIM), jnp.float32)
  agg2 = _sc_aggregate(x, src_t, dst_t, zeros_init)
  return _tc_linear(agg2, W1, b1, W2, b2)


# speedup vs baseline: 3.7883x; 1.1084x over previous
import functools

import jax
import jax.numpy as jnp
from jax import lax
from jax.experimental import pallas as pl
from jax.experimental.pallas import tpu as pltpu
from jax.experimental.pallas import tpu_sc as plsc

N_NODES = 10000
N_EDGES = 320000
DIM = 128

NC = 2
NS = 16
CHUNK = 128
CHUNKS_PER_TILE = 80
EDGES_PER_TILE = CHUNK * CHUNKS_PER_TILE
E_PAD = NC * NS * EDGES_PER_TILE
ACC_ROWS = 10112
ROWS_PER_TILE = ACC_ROWS // NS


def _sc_aggregate(x, idx_t, zeros_init):
  mesh = plsc.VectorSubcoreMesh(
      core_axis_name="c", subcore_axis_name="s", num_cores=NC,
      num_subcores=NS)

  NSTREAM = 2
  CPS = CHUNKS_PER_TILE // NSTREAM

  @functools.partial(
      pl.kernel,
      out_type=jax.ShapeDtypeStruct((NC, ACC_ROWS, DIM), jnp.float32),
      mesh=mesh,
      scratch_types=[
          pltpu.VMEM_SHARED((ACC_ROWS, DIM), jnp.float32),
          pltpu.VMEM((NSTREAM * 2, 2, CHUNK), jnp.int32),
          pltpu.VMEM((NSTREAM, CHUNK, DIM), jnp.float32),
          [pltpu.SemaphoreType.DMA] * NSTREAM,
          [pltpu.SemaphoreType.DMA] * NSTREAM,
          [pltpu.SemaphoreType.DMA] * NSTREAM,
      ],
  )
  def sc_kernel(x_hbm, idx_hbm, zer_hbm, out_hbm,
                acc, idxb, rows, gsems, ssems, isems):
    c = lax.axis_index("c")
    s = lax.axis_index("s")
    pltpu.sync_copy(zer_hbm.at[pl.ds(s * ROWS_PER_TILE, ROWS_PER_TILE)],
                    acc.at[pl.ds(s * ROWS_PER_TILE, ROWS_PER_TILE)])
    plsc.subcore_barrier()

    def fire_idx(t, i, pb):
      pltpu.async_copy(idx_hbm.at[c, s, t * CPS + i],
                       idxb.at[t * 2 + pb], isems[t])

    def wait_idx(t, pb):
      pltpu.make_async_copy(idx_hbm.at[0, 0, 0], idxb.at[t * 2 + pb],
                            isems[t]).wait()

    def fire_gather(t, pb):
      pltpu.async_copy(x_hbm.at[idxb.at[t * 2 + pb, 0]], rows.at[t],
                       gsems[t])

    def wait_gather(t, pb):
      pltpu.make_async_copy(x_hbm.at[idxb.at[t * 2 + pb, 0]],
                            rows.at[t], gsems[t]).wait()

    def fire_scatter(t, pb):
      pltpu.async_copy(rows.at[t], acc.at[idxb.at[t * 2 + pb, 1]],
                       ssems[t], add=True)

    def wait_scatter(t, pb):
      pltpu.make_async_copy(rows.at[t], acc.at[idxb.at[t * 2 + pb, 1]],
                            ssems[t]).wait()

    for t in range(NSTREAM):
      pltpu.sync_copy(idx_hbm.at[c, s, t * CPS], idxb.at[t * 2])
      fire_gather(t, 0)
      fire_idx(t, 1, 1)

    @pl.loop(0, CPS // 2)
    def _(j):
      for par in range(2):
        for t in range(NSTREAM):
          wait_gather(t, par)
          fire_scatter(t, par)
        for t in range(NSTREAM):
          wait_scatter(t, par)
          if par == 0:
            wait_idx(t, 1)
            fire_gather(t, 1)
            @pl.when(j < CPS // 2 - 1)
            def _():
              fire_idx(t, 2 * j + 2, 0)
          else:
            @pl.when(j < CPS // 2 - 1)
            def _():
              wait_idx(t, 0)
              fire_gather(t, 0)
              fire_idx(t, 2 * j + 3, 1)

    plsc.subcore_barrier()
    pltpu.sync_copy(
        acc.at[pl.ds(s * ROWS_PER_TILE, ROWS_PER_TILE)],
        out_hbm.at[c, pl.ds(s * ROWS_PER_TILE, ROWS_PER_TILE)])

  return sc_kernel(x, idx_t, zeros_init)


def _tc_body(a_ref, w1_ref, b1_ref, w2_ref, b2_ref, o_ref):
  a = a_ref[0, :N_NODES] + a_ref[1, :N_NODES]
  h = lax.dot_general(a, w1_ref[...], (((1,), (1,)), ((), ())),
                      preferred_element_type=jnp.float32)
  h = jnp.maximum(h + b1_ref[...], 0.0)
  o = lax.dot_general(w2_ref[...], h, (((1,), (1,)), ((), ())),
                      preferred_element_type=jnp.float32)
  o_ref[...] = o + b2_ref[...]


def _tc_linear(agg2, W1, b1, W2, b2):
  return pl.pallas_call(
      _tc_body,
      out_shape=jax.ShapeDtypeStruct((DIM, N_NODES), jnp.float32),
  )(agg2, W1, b1.reshape(1, DIM), W2, b2.reshape(DIM, 1))


def kernel(x, edge_index, W1, b1, W2, b2):
  src = edge_index[0]
  dst = edge_index[1]
  pad = E_PAD - N_EDGES
  src_p = jnp.concatenate([src, jnp.zeros((pad,), jnp.int32)])
  dst_p = jnp.concatenate(
      [dst, jnp.full((pad,), N_NODES, jnp.int32)])
  src_t = src_p.reshape(NC, NS, CHUNKS_PER_TILE, 1, CHUNK)
  dst_t = dst_p.reshape(NC, NS, CHUNKS_PER_TILE, 1, CHUNK)
  idx_t = jnp.concatenate([src_t, dst_t], axis=3)
  zeros_init = jnp.zeros((ACC_ROWS, DIM), jnp.float32)
  agg2 = _sc_aggregate(x, idx_t, zeros_init)
  return _tc_linear(agg2, W1, b1, W2, b2)
